# Initial kernel scaffold; baseline (speedup 1.0000x reference)
import jax
import jax.numpy as jnp
from jax.experimental import pallas as pl


def kernel(x, edge_index, W_self1, W_neigh1, b1, W_self2, W_neigh2, b2):
    raise NotImplementedError("write your pallas kernel here")


# speedup vs baseline: 3.8110x; 3.8110x over previous
import functools

import jax
import jax.numpy as jnp
from jax import lax
from jax.experimental import pallas as pl
from jax.experimental.pallas import tpu as pltpu
from jax.experimental.pallas import tpu_sc as plsc

N = 10000
D = 128
H = 128
NC = 2
NS = 16
NW = NC * NS
C = 128
NP = 10240
RPW = NP // NS

_MESH = plsc.VectorSubcoreMesh(
    core_axis_name="c", subcore_axis_name="s", num_cores=NC, num_subcores=NS
)


def _make_sc_agg(num_chunks: int):
    @functools.partial(
        pl.kernel,
        out_type=jax.ShapeDtypeStruct((NC, NP, D), jnp.float32),
        mesh=_MESH,
        scratch_types=[
            pltpu.VMEM((C,), jnp.int32),
            pltpu.VMEM((C,), jnp.int32),
            pltpu.VMEM((C, D), jnp.float32),
            pltpu.VMEM_SHARED((NP, D), jnp.float32),
            pltpu.SemaphoreType.DMA,
        ],
    )
    def sc_agg(h_hbm, srcs_hbm, dsts_hbm, zblk_hbm,
               agg_out, src_v, dst_v, rows_v, agg_sh, sem):
        cid = lax.axis_index("c")
        sid = lax.axis_index("s")
        wid = cid * NS + sid

        pltpu.sync_copy(zblk_hbm, rows_v)
        for r in range(RPW // C):
            pltpu.sync_copy(rows_v, agg_sh.at[pl.ds(sid * RPW + r * C, C)])
        plsc.subcore_barrier()

        def body(k, _):
            row = wid * num_chunks + k
            pltpu.sync_copy(srcs_hbm.at[row], src_v)
            pltpu.sync_copy(dsts_hbm.at[row], dst_v)
            pltpu.async_copy(h_hbm.at[src_v], rows_v, sem).wait()
            pltpu.sync_copy(rows_v, agg_sh.at[dst_v], add=True)
            return _

        lax.fori_loop(0, num_chunks, body, None)

        plsc.subcore_barrier()
        for r in range(RPW // C):
            off = sid * RPW + r * C
            pltpu.sync_copy(agg_sh.at[pl.ds(off, C)], rows_v)
            pltpu.sync_copy(rows_v, agg_out.at[cid, pl.ds(off, C)])

    return sc_agg


def _make_sc_deg(num_chunks: int):
    @functools.partial(
        pl.kernel,
        out_type=jax.ShapeDtypeStruct((NC, NP, D), jnp.float32),
        mesh=_MESH,
        scratch_types=[
            pltpu.VMEM((C,), jnp.int32),
            pltpu.VMEM((C, D), jnp.float32),
            pltpu.VMEM_SHARED((NP, D), jnp.float32),
        ],
    )
    def sc_deg(dsts_hbm, zblk_hbm, oblk_hbm,
               deg_out, dst_v, rows_v, deg_sh):
        cid = lax.axis_index("c")
        sid = lax.axis_index("s")
        wid = cid * NS + sid

        pltpu.sync_copy(zblk_hbm, rows_v)
        for r in range(RPW // C):
            pltpu.sync_copy(rows_v, deg_sh.at[pl.ds(sid * RPW + r * C, C)])
        pltpu.sync_copy(oblk_hbm, rows_v)
        plsc.subcore_barrier()

        def body(k, _):
            row = wid * num_chunks + k
            pltpu.sync_copy(dsts_hbm.at[row], dst_v)
            pltpu.sync_copy(rows_v, deg_sh.at[dst_v], add=True)
            return _

        lax.fori_loop(0, num_chunks, body, None)

        plsc.subcore_barrier()
        for r in range(RPW // C):
            off = sid * RPW + r * C
            pltpu.sync_copy(deg_sh.at[pl.ds(off, C)], rows_v)
            pltpu.sync_copy(rows_v, deg_out.at[cid, pl.ds(off, C)])

    return sc_deg


def _make_tc_layer(final: bool, bn: int = 1000):
    def body(x_ref, p_ref, deg_ref, ws_ref, wn_ref, b_ref, o_ref):
        xb = x_ref[...]
        pb = p_ref[0] + p_ref[1]
        deg = deg_ref[0, :, 0:1] + deg_ref[1, :, 0:1]
        hn = pb / jnp.maximum(deg, 1.0)
        acc = lax.dot_general(xb, ws_ref[...], (((1,), (1,)), ((), ())),
                              preferred_element_type=jnp.float32)
        acc = acc + lax.dot_general(hn, wn_ref[...], (((1,), (1,)), ((), ())),
                                    preferred_element_type=jnp.float32)
        acc = acc + b_ref[...]
        acc = jnp.maximum(acc, 0.0)
        if final:
            nrm = jnp.sqrt(jnp.sum(acc * acc, axis=1, keepdims=True))
            acc = acc / jnp.maximum(nrm, 1e-12)
        o_ref[...] = acc

    grid = N // bn
    return pl.pallas_call(
        body,
        grid=(grid,),
        in_specs=[
            pl.BlockSpec((bn, D), lambda i: (i, 0)),
            pl.BlockSpec((NC, bn, D), lambda i: (0, i, 0)),
            pl.BlockSpec((NC, bn, D), lambda i: (0, i, 0)),
            pl.BlockSpec((H, D), lambda i: (0, 0)),
            pl.BlockSpec((H, D), lambda i: (0, 0)),
            pl.BlockSpec((1, H), lambda i: (0, 0)),
        ],
        out_specs=pl.BlockSpec((bn, H), lambda i: (i, 0)),
        out_shape=jax.ShapeDtypeStruct((N, H), jnp.float32),
    )


def kernel(x, edge_index, W_self1, W_neigh1, b1, W_self2, W_neigh2, b2):
    e = edge_index.shape[1]
    num_chunks = -(-e // (NW * C))
    pad = NW * num_chunks * C - e

    src = edge_index[0]
    dst = edge_index[1]
    srcs = jnp.concatenate([src, jnp.zeros((pad,), jnp.int32)]).reshape(
        NW * num_chunks, C)
    dsts = jnp.concatenate([dst, jnp.full((pad,), N, jnp.int32)]).reshape(
        NW * num_chunks, C)

    zblk = jnp.zeros((C, D), jnp.float32)
    oblk = jnp.ones((C, D), jnp.float32)

    sc_agg = _make_sc_agg(num_chunks)
    sc_deg = _make_sc_deg(num_chunks)
    tc_layer1 = _make_tc_layer(final=False)
    tc_layer2 = _make_tc_layer(final=True)

    deg = sc_deg(dsts, zblk, oblk)
    agg1 = sc_agg(x, srcs, dsts, zblk)
    h1 = tc_layer1(x, agg1, deg, W_self1, W_neigh1, b1.reshape(1, H))
    agg2 = sc_agg(h1, srcs, dsts, zblk)
    out = tc_layer2(h1, agg2, deg, W_self2, W_neigh2, b2.reshape(1, H))
    return out
